# baseline (speedup 1.0000x reference)
import functools

import jax
import jax.numpy as jnp
from jax import lax
from jax.experimental import pallas as pl
from jax.experimental.pallas import tpu as pltpu
from jax.experimental.pallas import tpu_sc as plsc

N = 10000
D = 128
NC = 2
NS = 16
NT = NC * NS
CH = 128
NCH = 80
EPAD = NT * NCH * CH
NPAD = 10240
RPT = NPAD // NS
ZR = 16

_f32 = jnp.float32


@functools.cache
def _mesh():
    return plsc.VectorSubcoreMesh(
        core_axis_name="c", subcore_axis_name="s",
        num_cores=NC, num_subcores=NS)
_zeros16 = None


def _deg_body(dst_hbm, out_hbm, idxv, onesv, zv, acc, sem):
    cid = lax.axis_index("c")
    sid = lax.axis_index("s")
    wid = cid * NS + sid
    z16 = jnp.zeros((16,), _f32)
    for j in range(8):
        onesv[pl.ds(j * 16, 16)] = jnp.full((16,), 1.0, _f32)

    def zfill(i, _):
        zv[pl.ds(i * 16, 16)] = z16
        return 0
    lax.fori_loop(0, RPT // 16, zfill, 0)

    pltpu.sync_copy(dst_hbm.at[wid], idxv)
    pltpu.sync_copy(zv, acc.at[pl.ds(sid * RPT, RPT)])
    plsc.subcore_barrier()

    def body(j, _):
        pltpu.sync_copy(onesv, acc.at[idxv.at[j]], add=True)
        return 0
    lax.fori_loop(0, NCH, body, 0)
    plsc.subcore_barrier()
    pltpu.sync_copy(acc.at[pl.ds(sid * RPT, RPT)],
                    out_hbm.at[cid, pl.ds(sid * RPT, RPT)])


@functools.cache
def _deg_kernel():
    return pl.kernel(
        _deg_body,
        out_type=jax.ShapeDtypeStruct((NC, NPAD), _f32),
        mesh=_mesh(),
        scratch_types=[
            pltpu.VMEM((NCH, CH), jnp.int32),
            pltpu.VMEM((CH,), _f32),
            pltpu.VMEM((RPT,), _f32),
            pltpu.VMEM_SHARED((NPAD,), _f32),
            pltpu.SemaphoreType.DMA,
        ],
    )


NB = 2
P = 32
CPT = 160


def _pass_body(table_hbm, src_hbm, dst_hbm, out_hbm,
               srcv, dstv, r0, r1, zbuf, acc, s0, s1):
    cid = lax.axis_index("c")
    sid = lax.axis_index("s")
    rows = [r0, r1]
    sems = [s0, s1]
    z16 = jnp.zeros((16,), _f32)

    @pl.when(cid == 0)
    def _():
        def zfill(i, _):
            for j in range(D // 16):
                zbuf[i, pl.ds(j * 16, 16)] = z16
            return 0
        lax.fori_loop(0, ZR, zfill, 0)

        def zacc(r, _):
            pltpu.sync_copy(zbuf, acc.at[pl.ds(sid * RPT + r * ZR, ZR)])
            return 0
        lax.fori_loop(0, RPT // ZR, zacc, 0)
        plsc.subcore_barrier()

        for st in range(CPT // P):
            pltpu.sync_copy(src_hbm.at[sid, pl.ds(st * P, P)], srcv)
            pltpu.sync_copy(dst_hbm.at[sid, pl.ds(st * P, P)], dstv)
            for b in range(NB):
                pltpu.async_copy(table_hbm.at[srcv.at[b]], rows[b], sems[b])

            def body(i, _):
                for b in range(NB):
                    j = i * NB + b
                    pltpu.make_async_copy(
                        table_hbm.at[srcv.at[j]], rows[b], sems[b]).wait()
                    pltpu.sync_copy(rows[b], acc.at[dstv.at[j]], add=True)
                    jn = j + NB

                    @pl.when(jn < P)
                    def _():
                        pltpu.async_copy(
                            table_hbm.at[srcv.at[jn]], rows[b], sems[b])
                return 0
            lax.fori_loop(0, P // NB, body, 0)

        plsc.subcore_barrier()
        pltpu.sync_copy(acc.at[pl.ds(sid * RPT, RPT)],
                        out_hbm.at[pl.ds(sid * RPT, RPT), :])


@functools.cache
def _pass_kernel():
    return pl.kernel(
        _pass_body,
        out_type=jax.ShapeDtypeStruct((NPAD, D), _f32),
        mesh=_mesh(),
        scratch_types=[
            pltpu.VMEM((P, CH), jnp.int32),
            pltpu.VMEM((P, CH), jnp.int32),
            pltpu.VMEM((CH, D), _f32),
            pltpu.VMEM((CH, D), _f32),
            pltpu.VMEM((ZR, D), _f32),
            pltpu.VMEM_SHARED((NPAD, D), _f32),
            pltpu.SemaphoreType.DMA,
            pltpu.SemaphoreType.DMA,
        ],
    )


def _scale_body(x_ref, c_ref, o_ref):
    o_ref[...] = x_ref[...] * c_ref[...]


def _scale_rows(x, col, blk=400):
    m = x.shape[0] // blk
    return pl.pallas_call(
        _scale_body,
        grid=(m,),
        in_specs=[
            pl.BlockSpec((blk, D), lambda i: (i, 0)),
            pl.BlockSpec((blk, 1), lambda i: (i, 0)),
        ],
        out_specs=pl.BlockSpec((blk, D), lambda i: (i, 0)),
        out_shape=jax.ShapeDtypeStruct((x.shape[0], D), _f32),
    )(x, col)


def _final_body(x_ref, g2_ref, a2_ref, sc_ref, nc_ref, w_ref, b_ref, o_ref):
    x = x_ref[...]
    h1 = g2_ref[...] * sc_ref[...]
    h2 = a2_ref[...] * nc_ref[...]
    w = w_ref[...]
    acc = jnp.dot(x, w[0], preferred_element_type=_f32)
    acc += jnp.dot(h1, w[1], preferred_element_type=_f32)
    acc += jnp.dot(h2, w[2], preferred_element_type=_f32)
    o_ref[...] = acc + b_ref[...]


def _final(feat, g2, acc2, scol, ncol, wcat, bias, blk=400):
    m = N // blk
    return pl.pallas_call(
        _final_body,
        grid=(m,),
        in_specs=[
            pl.BlockSpec((blk, D), lambda i: (i, 0)),
            pl.BlockSpec((blk, D), lambda i: (i, 0)),
            pl.BlockSpec((blk, D), lambda i: (i, 0)),
            pl.BlockSpec((blk, 1), lambda i: (i, 0)),
            pl.BlockSpec((blk, 1), lambda i: (i, 0)),
            pl.BlockSpec((3, D, D), lambda i: (0, 0, 0)),
            pl.BlockSpec((1, D), lambda i: (0, 0)),
        ],
        out_specs=pl.BlockSpec((blk, D), lambda i: (i, 0)),
        out_shape=jax.ShapeDtypeStruct((N, D), _f32),
    )(feat, g2, acc2, scol, ncol, wcat, bias)


@jax.jit
def kernel(feat, edge_index, W, bias):
    src = edge_index[0]
    dst = edge_index[1]
    pad = EPAD - src.shape[0]
    srcp = jnp.concatenate([src, jnp.zeros((pad,), jnp.int32)])
    dstp = jnp.concatenate([dst, jnp.full((pad,), N, jnp.int32)])
    srcA = srcp.reshape(NS, CPT, CH)
    dstA = dstp.reshape(NS, CPT, CH)

    deg2 = _deg_kernel()(dstp.reshape(NT, NCH, CH))
    degf = deg2[0] + deg2[1]
    cdeg = jnp.clip(degf, 1.0, None)
    ncol_pad = cdeg ** -0.5
    n2col = jnp.reshape(1.0 / cdeg, (NPAD, 1))
    ncol = jnp.reshape(ncol_pad[:N], (N, 1))
    scol = jnp.reshape(cdeg[:N] ** 0.5, (N, 1))

    g1 = _scale_rows(feat, ncol)
    acc1 = _pass_kernel()(g1, srcA, dstA)
    g2 = _scale_rows(acc1, n2col, blk=512)
    acc2 = _pass_kernel()(g2, srcA, dstA)

    wcat = jnp.stack([
        (W[0] - W[2]).T,
        -W[1].T,
        2.0 * W[2].T,
    ])
    return _final(feat, g2, acc2, scol, ncol, wcat,
                  jnp.reshape(bias, (1, D)))


# speedup vs baseline: 1.0360x; 1.0360x over previous
import functools

import jax
import jax.numpy as jnp
from jax import lax
from jax.experimental import pallas as pl
from jax.experimental.pallas import tpu as pltpu
from jax.experimental.pallas import tpu_sc as plsc

N = 10000
D = 128
NC = 2
NS = 16
NT = NC * NS
CH = 128
NCH = 80
EPAD = NT * NCH * CH
NPAD = 10240
RPT = NPAD // NS
ZR = 16
NB = 2
HALF = NCH // 2

_f32 = jnp.float32


@functools.cache
def _mesh():
    return plsc.VectorSubcoreMesh(
        core_axis_name="c", subcore_axis_name="s",
        num_cores=NC, num_subcores=NS)


def _deg_body(dst_hbm, out_hbm, idxv, onesv, zv, acc, sem):
    cid = lax.axis_index("c")
    sid = lax.axis_index("s")
    wid = cid * NS + sid
    z16 = jnp.zeros((16,), _f32)
    for j in range(8):
        onesv[pl.ds(j * 16, 16)] = jnp.full((16,), 1.0, _f32)

    def zfill(i, _):
        zv[pl.ds(i * 16, 16)] = z16
        return 0
    lax.fori_loop(0, RPT // 16, zfill, 0)

    pltpu.sync_copy(dst_hbm.at[wid], idxv)
    pltpu.sync_copy(zv, acc.at[pl.ds(sid * RPT, RPT)])
    plsc.subcore_barrier()

    def body(j, _):
        pltpu.sync_copy(onesv, acc.at[idxv.at[j]], add=True)
        return 0
    lax.fori_loop(0, NCH, body, 0)
    plsc.subcore_barrier()
    pltpu.sync_copy(acc.at[pl.ds(sid * RPT, RPT)],
                    out_hbm.at[cid, pl.ds(sid * RPT, RPT)])


@functools.cache
def _deg_kernel():
    return pl.kernel(
        _deg_body,
        out_type=jax.ShapeDtypeStruct((NC, NPAD), _f32),
        mesh=_mesh(),
        scratch_types=[
            pltpu.VMEM((NCH, CH), jnp.int32),
            pltpu.VMEM((CH,), _f32),
            pltpu.VMEM((RPT,), _f32),
            pltpu.VMEM_SHARED((NPAD,), _f32),
            pltpu.SemaphoreType.DMA,
        ],
    )


def _pass_body(table_hbm, src_hbm, dst_hbm, out_hbm,
               srcv, dstv, r0, r1, zbuf, acc, s0, s1):
    cid = lax.axis_index("c")
    sid = lax.axis_index("s")
    wid = cid * NS + sid
    rows = [r0, r1]
    sems = [s0, s1]
    z16 = jnp.zeros((16,), _f32)

    def zfill(i, _):
        for j in range(D // 16):
            zbuf[i, pl.ds(j * 16, 16)] = z16
        return 0
    lax.fori_loop(0, ZR, zfill, 0)

    def zacc(r, _):
        pltpu.sync_copy(zbuf, acc.at[pl.ds(sid * RPT + r * ZR, ZR)])
        return 0
    lax.fori_loop(0, RPT // ZR, zacc, 0)
    plsc.subcore_barrier()

    for h in range(2):
        pltpu.sync_copy(src_hbm.at[wid, pl.ds(h * HALF, HALF)], srcv)
        pltpu.sync_copy(dst_hbm.at[wid, pl.ds(h * HALF, HALF)], dstv)
        for b in range(NB):
            pltpu.async_copy(table_hbm.at[srcv.at[b]], rows[b], sems[b])

        def body(i, _):
            for b in range(NB):
                j = i * NB + b
                pltpu.make_async_copy(
                    table_hbm.at[srcv.at[j]], rows[b], sems[b]).wait()
                pltpu.sync_copy(rows[b], acc.at[dstv.at[j]], add=True)
                jn = j + NB

                @pl.when(jn < HALF)
                def _():
                    pltpu.async_copy(
                        table_hbm.at[srcv.at[jn]], rows[b], sems[b])
            return 0
        lax.fori_loop(0, HALF // NB, body, 0)
    plsc.subcore_barrier()
    pltpu.sync_copy(acc.at[pl.ds(sid * RPT, RPT)],
                    out_hbm.at[cid, pl.ds(sid * RPT, RPT), :])


@functools.cache
def _pass_kernel():
    return pl.kernel(
        _pass_body,
        out_type=jax.ShapeDtypeStruct((NC, NPAD, D), _f32),
        mesh=_mesh(),
        scratch_types=[
            pltpu.VMEM((HALF, CH), jnp.int32),
            pltpu.VMEM((HALF, CH), jnp.int32),
            pltpu.VMEM((CH, D), _f32),
            pltpu.VMEM((CH, D), _f32),
            pltpu.VMEM((ZR, D), _f32),
            pltpu.VMEM_SHARED((NPAD, D), _f32),
            pltpu.SemaphoreType.DMA,
            pltpu.SemaphoreType.DMA,
        ],
    )


def _scale_body(x_ref, c_ref, o_ref):
    o_ref[...] = x_ref[...] * c_ref[...]


def _scale_rows(x, col, blk=400):
    m = x.shape[0] // blk
    return pl.pallas_call(
        _scale_body,
        grid=(m,),
        in_specs=[
            pl.BlockSpec((blk, D), lambda i: (i, 0)),
            pl.BlockSpec((blk, 1), lambda i: (i, 0)),
        ],
        out_specs=pl.BlockSpec((blk, D), lambda i: (i, 0)),
        out_shape=jax.ShapeDtypeStruct((x.shape[0], D), _f32),
    )(x, col)


def _scale2_body(a_ref, c_ref, o_ref):
    o_ref[...] = (a_ref[0] + a_ref[1]) * c_ref[...]


def _combine_scale(acc2, col, blk=512):
    m = acc2.shape[1] // blk
    return pl.pallas_call(
        _scale2_body,
        grid=(m,),
        in_specs=[
            pl.BlockSpec((NC, blk, D), lambda i: (0, i, 0)),
            pl.BlockSpec((blk, 1), lambda i: (i, 0)),
        ],
        out_specs=pl.BlockSpec((blk, D), lambda i: (i, 0)),
        out_shape=jax.ShapeDtypeStruct((acc2.shape[1], D), _f32),
    )(acc2, col)


def _final_body(x_ref, g2_ref, a2_ref, sc_ref, nc_ref, w_ref, b_ref, o_ref):
    x = x_ref[...]
    h1 = g2_ref[...] * sc_ref[...]
    h2 = (a2_ref[0] + a2_ref[1]) * nc_ref[...]
    w = w_ref[...]
    acc = jnp.dot(x, w[0], preferred_element_type=_f32)
    acc += jnp.dot(h1, w[1], preferred_element_type=_f32)
    acc += jnp.dot(h2, w[2], preferred_element_type=_f32)
    o_ref[...] = acc + b_ref[...]


def _final(feat, g2, acc2, scol, ncol, wcat, bias, blk=400):
    m = N // blk
    return pl.pallas_call(
        _final_body,
        grid=(m,),
        in_specs=[
            pl.BlockSpec((blk, D), lambda i: (i, 0)),
            pl.BlockSpec((blk, D), lambda i: (i, 0)),
            pl.BlockSpec((NC, blk, D), lambda i: (0, i, 0)),
            pl.BlockSpec((blk, 1), lambda i: (i, 0)),
            pl.BlockSpec((blk, 1), lambda i: (i, 0)),
            pl.BlockSpec((3, D, D), lambda i: (0, 0, 0)),
            pl.BlockSpec((1, D), lambda i: (0, 0)),
        ],
        out_specs=pl.BlockSpec((blk, D), lambda i: (i, 0)),
        out_shape=jax.ShapeDtypeStruct((N, D), _f32),
    )(feat, g2, acc2, scol, ncol, wcat, bias)


@jax.jit
def kernel(feat, edge_index, W, bias):
    src = edge_index[0]
    dst = edge_index[1]
    pad = EPAD - src.shape[0]
    pad_dst = N + (jnp.arange(pad, dtype=jnp.int32) % (NPAD - N))
    srcp = jnp.concatenate([src, jnp.zeros((pad,), jnp.int32)]).reshape(
        NT, NCH, CH)
    dstp = jnp.concatenate([dst, pad_dst]).reshape(NT, NCH, CH)

    deg2 = _deg_kernel()(dstp)
    degf = deg2[0] + deg2[1]
    cdeg = jnp.clip(degf, 1.0, None)
    n2col = jnp.reshape(1.0 / cdeg, (NPAD, 1))
    ncol = jnp.reshape(cdeg[:N] ** -0.5, (N, 1))
    scol = jnp.reshape(cdeg[:N] ** 0.5, (N, 1))

    g1 = _scale_rows(feat, ncol)
    acc1 = _pass_kernel()(g1, srcp, dstp)
    g2 = _combine_scale(acc1, n2col)
    acc2 = _pass_kernel()(g2, srcp, dstp)

    wcat = jnp.stack([
        (W[0] - W[2]).T,
        -W[1].T,
        2.0 * W[2].T,
    ])
    return _final(feat, g2, acc2, scol, ncol, wcat,
                  jnp.reshape(bias, (1, D)))


# speedup vs baseline: 1.1810x; 1.1400x over previous
import functools

import jax
import jax.numpy as jnp
from jax import lax
from jax.experimental import pallas as pl
from jax.experimental.pallas import tpu as pltpu
from jax.experimental.pallas import tpu_sc as plsc

N = 10000
D = 128
NC = 2
NS = 16
NT = NC * NS
CH = 128
NCH = 80
EPAD = NT * NCH * CH
NPAD = 10240
RPT = NPAD // NS
ZR = 16
NB = 2
P = 32
C0 = 128
C1 = 32

_f32 = jnp.float32


@functools.cache
def _mesh():
    return plsc.VectorSubcoreMesh(
        core_axis_name="c", subcore_axis_name="s",
        num_cores=NC, num_subcores=NS)


def _deg_body(dst_hbm, out_hbm, idxv, onesv, zv, acc, sem):
    cid = lax.axis_index("c")
    sid = lax.axis_index("s")
    wid = cid * NS + sid
    z16 = jnp.zeros((16,), _f32)
    for j in range(8):
        onesv[pl.ds(j * 16, 16)] = jnp.full((16,), 1.0, _f32)

    def zfill(i, _):
        zv[pl.ds(i * 16, 16)] = z16
        return 0
    lax.fori_loop(0, RPT // 16, zfill, 0)

    pltpu.sync_copy(dst_hbm.at[wid], idxv)
    pltpu.sync_copy(zv, acc.at[pl.ds(sid * RPT, RPT)])
    plsc.subcore_barrier()

    def body(j, _):
        pltpu.sync_copy(onesv, acc.at[idxv.at[j]], add=True)
        return 0
    lax.fori_loop(0, NCH, body, 0)
    plsc.subcore_barrier()
    pltpu.sync_copy(acc.at[pl.ds(sid * RPT, RPT)],
                    out_hbm.at[cid, pl.ds(sid * RPT, RPT)])


@functools.cache
def _deg_kernel():
    return pl.kernel(
        _deg_body,
        out_type=jax.ShapeDtypeStruct((NC, NPAD), _f32),
        mesh=_mesh(),
        scratch_types=[
            pltpu.VMEM((NCH, CH), jnp.int32),
            pltpu.VMEM((CH,), _f32),
            pltpu.VMEM((RPT,), _f32),
            pltpu.VMEM_SHARED((NPAD,), _f32),
            pltpu.SemaphoreType.DMA,
        ],
    )


def _pass_body(table_hbm, srcA, dstA, srcB, dstB, out_hbm,
               srcv, dstv, r0, r1, zbuf, acc, s0, s1):
    cid = lax.axis_index("c")
    sid = lax.axis_index("s")
    rows = [r0, r1]
    sems = [s0, s1]
    z16 = jnp.zeros((16,), _f32)

    def zfill(i, _):
        for j in range(D // 16):
            zbuf[i, pl.ds(j * 16, 16)] = z16
        return 0
    lax.fori_loop(0, ZR, zfill, 0)

    def zacc(r, _):
        pltpu.sync_copy(zbuf, acc.at[pl.ds(sid * RPT + r * ZR, ZR)])
        return 0
    lax.fori_loop(0, RPT // ZR, zacc, 0)
    plsc.subcore_barrier()

    def run(src_hbm, dst_hbm, nch):
        for st in range(nch // P):
            pltpu.sync_copy(src_hbm.at[sid, pl.ds(st * P, P)], srcv)
            pltpu.sync_copy(dst_hbm.at[sid, pl.ds(st * P, P)], dstv)
            for b in range(NB):
                pltpu.async_copy(table_hbm.at[srcv.at[b]], rows[b], sems[b])

            def body(i, _):
                for b in range(NB):
                    j = i * NB + b
                    pltpu.make_async_copy(
                        table_hbm.at[srcv.at[j]], rows[b], sems[b]).wait()
                    pltpu.sync_copy(rows[b], acc.at[dstv.at[j]], add=True)
                    jn = j + NB

                    @pl.when(jn < P)
                    def _():
                        pltpu.async_copy(
                            table_hbm.at[srcv.at[jn]], rows[b], sems[b])
                return 0
            lax.fori_loop(0, P // NB, body, 0)

    @pl.when(cid == 0)
    def _():
        run(srcA, dstA, C0)

    @pl.when(cid == 1)
    def _():
        run(srcB, dstB, C1)

    plsc.subcore_barrier()
    pltpu.sync_copy(acc.at[pl.ds(sid * RPT, RPT)],
                    out_hbm.at[cid, pl.ds(sid * RPT, RPT), :])


@functools.cache
def _pass_kernel():
    return pl.kernel(
        _pass_body,
        out_type=jax.ShapeDtypeStruct((NC, NPAD, D), _f32),
        mesh=_mesh(),
        scratch_types=[
            pltpu.VMEM((P, CH), jnp.int32),
            pltpu.VMEM((P, CH), jnp.int32),
            pltpu.VMEM((CH, D), _f32),
            pltpu.VMEM((CH, D), _f32),
            pltpu.VMEM((ZR, D), _f32),
            pltpu.VMEM_SHARED((NPAD, D), _f32),
            pltpu.SemaphoreType.DMA,
            pltpu.SemaphoreType.DMA,
        ],
    )


def _scale_body(x_ref, c_ref, o_ref):
    o_ref[...] = x_ref[...] * c_ref[...]


def _scale_rows(x, col, blk=400):
    m = x.shape[0] // blk
    return pl.pallas_call(
        _scale_body,
        grid=(m,),
        in_specs=[
            pl.BlockSpec((blk, D), lambda i: (i, 0)),
            pl.BlockSpec((blk, 1), lambda i: (i, 0)),
        ],
        out_specs=pl.BlockSpec((blk, D), lambda i: (i, 0)),
        out_shape=jax.ShapeDtypeStruct((x.shape[0], D), _f32),
    )(x, col)


def _scale2_body(a_ref, c_ref, o_ref):
    o_ref[...] = (a_ref[0] + a_ref[1]) * c_ref[...]


def _combine_scale(acc2, col, blk=512):
    m = acc2.shape[1] // blk
    return pl.pallas_call(
        _scale2_body,
        grid=(m,),
        in_specs=[
            pl.BlockSpec((NC, blk, D), lambda i: (0, i, 0)),
            pl.BlockSpec((blk, 1), lambda i: (i, 0)),
        ],
        out_specs=pl.BlockSpec((blk, D), lambda i: (i, 0)),
        out_shape=jax.ShapeDtypeStruct((acc2.shape[1], D), _f32),
    )(acc2, col)


def _final_body(x_ref, g2_ref, a2_ref, sc_ref, nc_ref, w_ref, b_ref, o_ref):
    x = x_ref[...]
    h1 = g2_ref[...] * sc_ref[...]
    h2 = (a2_ref[0] + a2_ref[1]) * nc_ref[...]
    w = w_ref[...]
    acc = jnp.dot(x, w[0], preferred_element_type=_f32)
    acc += jnp.dot(h1, w[1], preferred_element_type=_f32)
    acc += jnp.dot(h2, w[2], preferred_element_type=_f32)
    o_ref[...] = acc + b_ref[...]


def _final(feat, g2, acc2, scol, ncol, wcat, bias, blk=400):
    m = N // blk
    return pl.pallas_call(
        _final_body,
        grid=(m,),
        in_specs=[
            pl.BlockSpec((blk, D), lambda i: (i, 0)),
            pl.BlockSpec((blk, D), lambda i: (i, 0)),
            pl.BlockSpec((NC, blk, D), lambda i: (0, i, 0)),
            pl.BlockSpec((blk, 1), lambda i: (i, 0)),
            pl.BlockSpec((blk, 1), lambda i: (i, 0)),
            pl.BlockSpec((3, D, D), lambda i: (0, 0, 0)),
            pl.BlockSpec((1, D), lambda i: (0, 0)),
        ],
        out_specs=pl.BlockSpec((blk, D), lambda i: (i, 0)),
        out_shape=jax.ShapeDtypeStruct((N, D), _f32),
    )(feat, g2, acc2, scol, ncol, wcat, bias)


@jax.jit
def kernel(feat, edge_index, W, bias):
    src = edge_index[0]
    dst = edge_index[1]
    pad = EPAD - src.shape[0]
    pad_dst = N + (jnp.arange(pad, dtype=jnp.int32) % (NPAD - N))
    srcp = jnp.concatenate([src, jnp.zeros((pad,), jnp.int32)])
    dstp = jnp.concatenate([dst, pad_dst])
    eA = NS * C0 * CH
    srcA = srcp[:eA].reshape(NS, C0, CH)
    dstA = dstp[:eA].reshape(NS, C0, CH)
    srcB = srcp[eA:].reshape(NS, C1, CH)
    dstB = dstp[eA:].reshape(NS, C1, CH)

    deg2 = _deg_kernel()(dstp.reshape(NT, NCH, CH))
    degf = deg2[0] + deg2[1]
    cdeg = jnp.clip(degf, 1.0, None)
    n2col = jnp.reshape(1.0 / cdeg, (NPAD, 1))
    ncol = jnp.reshape(cdeg[:N] ** -0.5, (N, 1))
    scol = jnp.reshape(cdeg[:N] ** 0.5, (N, 1))

    g1 = _scale_rows(feat, ncol)
    acc1 = _pass_kernel()(g1, srcA, dstA, srcB, dstB)
    g2 = _combine_scale(acc1, n2col)
    acc2 = _pass_kernel()(g2, srcA, dstA, srcB, dstB)

    wcat = jnp.stack([
        (W[0] - W[2]).T,
        -W[1].T,
        2.0 * W[2].T,
    ])
    return _final(feat, g2, acc2, scol, ncol, wcat,
                  jnp.reshape(bias, (1, D)))
